# baseline (speedup 1.0000x reference)
import functools

import jax
import jax.numpy as jnp
from jax import lax
from jax.experimental import pallas as pl
from jax.experimental.pallas import tpu as pltpu
from jax.experimental.pallas import tpu_sc as plsc

LANE = 16


def _tc_node_proj(x, w_st):
    n = x.shape[0]
    bn = 1000

    def body(x_ref, w_ref, ps_ref, pt_ref):
        p = jnp.dot(x_ref[...], w_ref[...], preferred_element_type=jnp.float32)
        ps_ref[...] = p[:, :128]
        pt_ref[...] = p[:, 128:]

    return pl.pallas_call(
        body,
        grid=(n // bn,),
        in_specs=[
            pl.BlockSpec((bn, 128), lambda i: (i, 0)),
            pl.BlockSpec((128, 256), lambda i: (0, 0)),
        ],
        out_specs=[
            pl.BlockSpec((bn, 128), lambda i: (i, 0)),
            pl.BlockSpec((bn, 128), lambda i: (i, 0)),
        ],
        out_shape=[jax.ShapeDtypeStruct((n, 128), jnp.float32)] * 2,
    )(x, w_st)


def _tc_final(x, agg0, agg1, deg0, deg1, eW2, eb2, nw1a, nw1b, nb1, nw2, nb2):
    n = x.shape[0]
    bn = 1000

    def body(x_ref, a0_ref, a1_ref, d0_ref, d1_ref, w2_ref, b2e_ref, wa_ref,
             wb_ref, b1_ref, w3_ref, b2_ref, out_ref):
        deg = d0_ref[...][:, :1] + d1_ref[...][:, :1]
        agg = (
            jnp.dot(a0_ref[...] + a1_ref[...], w2_ref[...],
                    preferred_element_type=jnp.float32)
            + deg * b2e_ref[...]
        )
        pre = (
            jnp.dot(x_ref[...], wa_ref[...], preferred_element_type=jnp.float32)
            + jnp.dot(agg, wb_ref[...], preferred_element_type=jnp.float32)
            + b1_ref[...]
        )
        h2 = jnp.maximum(pre, 0.0)
        out_ref[...] = (
            x_ref[...]
            + jnp.dot(h2, w3_ref[...], preferred_element_type=jnp.float32)
            + b2_ref[...]
        )

    return pl.pallas_call(
        body,
        grid=(n // bn,),
        in_specs=[
            pl.BlockSpec((bn, 128), lambda i: (i, 0)),
            pl.BlockSpec((bn, 128), lambda i: (i, 0)),
            pl.BlockSpec((bn, 128), lambda i: (i, 0)),
            pl.BlockSpec((bn, 128), lambda i: (i, 0)),
            pl.BlockSpec((bn, 128), lambda i: (i, 0)),
            pl.BlockSpec((128, 128), lambda i: (0, 0)),
            pl.BlockSpec((1, 128), lambda i: (0, 0)),
            pl.BlockSpec((128, 128), lambda i: (0, 0)),
            pl.BlockSpec((128, 128), lambda i: (0, 0)),
            pl.BlockSpec((1, 128), lambda i: (0, 0)),
            pl.BlockSpec((128, 128), lambda i: (0, 0)),
            pl.BlockSpec((1, 128), lambda i: (0, 0)),
        ],
        out_specs=pl.BlockSpec((bn, 128), lambda i: (i, 0)),
        out_shape=jax.ShapeDtypeStruct((n, 128), jnp.float32),
    )(x, agg0, agg1, deg0, deg1, eW2, eb2, nw1a, nw1b, nb1, nw2, nb2)


def _sc_edge_agg(ps, pt, ef_flat, w_flat, src, tgt):
    n = ps.shape[0]
    e = src.shape[0]
    nc, ns = 2, 16
    nw = nc * ns
    epw = e // nw
    bsz = 40
    nb = epw // bsz
    npad = ((n + 8 * ns - 1) // (8 * ns)) * (8 * ns)
    rpt = npad // ns
    mesh = plsc.VectorSubcoreMesh(core_axis_name="c", subcore_axis_name="s")

    @functools.partial(
        pl.kernel,
        mesh=mesh,
        out_type=jax.ShapeDtypeStruct((nc, npad, 128), jnp.float32),
        scratch_types=[
            pltpu.VMEM((bsz,), jnp.int32),
            pltpu.VMEM((bsz,), jnp.int32),
            pltpu.VMEM((bsz,), jnp.int32),
            pltpu.VMEM((bsz,), jnp.int32),
            pltpu.VMEM((bsz, 128), jnp.float32),
            pltpu.VMEM((bsz, 128), jnp.float32),
            pltpu.VMEM((bsz, 128), jnp.float32),
            pltpu.VMEM((bsz, 128), jnp.float32),
            pltpu.VMEM((4 * bsz + LANE,), jnp.float32),
            pltpu.VMEM((4 * bsz + LANE,), jnp.float32),
            pltpu.VMEM((640,), jnp.float32),
            pltpu.VMEM((bsz, 128), jnp.float32),
            pltpu.SemaphoreType.DMA,
            pltpu.SemaphoreType.DMA,
            pltpu.SemaphoreType.DMA,
            pltpu.SemaphoreType.DMA,
            pltpu.VMEM_SHARED((npad, 128), jnp.float32),
        ],
    )
    def k(ps_hbm, pt_hbm, ef_hbm, w_hbm, src_hbm, tgt_hbm, out_hbm,
          sid0, sid1, tid0, tid1, ps0, ps1, pt0, pt1, ef0, ef1, wtab, hv,
          semi0, semi1, semg0, semg1, accum):
        c = lax.axis_index("c")
        s = lax.axis_index("s")
        wid = c * ns + s
        sidb = (sid0, sid1)
        tidb = (tid0, tid1)
        psb = (ps0, ps1)
        ptb = (pt0, pt1)
        efb = (ef0, ef1)
        semi = (semi0, semi1)
        semg = (semg0, semg1)

        zero16 = jnp.zeros((LANE,), jnp.float32)

        def zrow(r, carry):
            for j in range(128 // LANE):
                ps0[r, pl.ds(j * LANE, LANE)] = zero16
            return carry

        lax.fori_loop(0, bsz, zrow, 0)
        pltpu.sync_copy(w_hbm, wtab)

        nfull, rem = rpt // bsz, rpt % bsz
        for zi in range(nfull):
            pltpu.sync_copy(ps0, accum.at[pl.ds(s * rpt + zi * bsz, bsz)])
        if rem:
            pltpu.sync_copy(ps0.at[pl.ds(0, rem)],
                            accum.at[pl.ds(s * rpt + nfull * bsz, rem)])
        plsc.subcore_barrier()

        base = wid * epw

        def issue_idx(g, p):
            off = base + g * bsz
            pltpu.async_copy(src_hbm.at[pl.ds(off, bsz)], sidb[p], semi[p])
            pltpu.async_copy(tgt_hbm.at[pl.ds(off, bsz)], tidb[p], semi[p])

        def drain_idx(p):
            pltpu.make_async_copy(src_hbm.at[pl.ds(0, bsz)], sidb[p],
                                  semi[p]).wait()
            pltpu.make_async_copy(tgt_hbm.at[pl.ds(0, bsz)], tidb[p],
                                  semi[p]).wait()

        def issue_gathers(g, p):
            off = base + g * bsz
            pltpu.async_copy(ps_hbm.at[sidb[p]], psb[p], semg[p])
            pltpu.async_copy(pt_hbm.at[tidb[p]], ptb[p], semg[p])
            pltpu.async_copy(ef_hbm.at[pl.ds(4 * off, 4 * bsz)],
                             efb[p].at[pl.ds(0, 4 * bsz)], semg[p])

        def drain_gathers(p):
            for dst in (psb[p], ptb[p]):
                pltpu.make_async_copy(ps_hbm.at[pl.ds(0, bsz)], dst,
                                      semg[p]).wait()
            pltpu.make_async_copy(ef_hbm.at[pl.ds(0, 4 * bsz)],
                                  efb[p].at[pl.ds(0, 4 * bsz)],
                                  semg[p]).wait()

        bidx = [jnp.full((LANE,), kk, jnp.int32) for kk in range(4)]

        def process(p):
            def row(r, rc):
                efc = efb[p][pl.ds(4 * r, LANE)]
                eb = [efc.at[bidx[kk]].get(mode="promise_in_bounds")
                      for kk in range(4)]
                for j in range(128 // LANE):
                    sl = pl.ds(j * LANE, LANE)
                    v = psb[p][r, sl] + ptb[p][r, sl] + wtab[pl.ds(512 + j * LANE, LANE)]
                    for kk in range(4):
                        v = v + eb[kk] * wtab[pl.ds(kk * 128 + j * LANE, LANE)]
                    hv[r, sl] = jnp.maximum(v, 0.0)
                return rc

            lax.fori_loop(0, bsz, row, 0)
            pltpu.sync_copy(hv, accum.at[tidb[p]], add=True)

        pltpu.sync_copy(src_hbm.at[pl.ds(base, bsz)], sid0)
        pltpu.sync_copy(tgt_hbm.at[pl.ds(base, bsz)], tid0)
        issue_gathers(0, 0)
        issue_idx(1, 1)

        def pipe(i, carry):
            g1 = 2 * i + 1
            g2 = jnp.minimum(2 * i + 2, nb - 1)
            g3 = jnp.minimum(2 * i + 3, nb - 1)
            drain_idx(1)
            issue_gathers(g1, 1)
            drain_gathers(0)
            process(0)
            issue_idx(g2, 0)
            drain_idx(0)
            issue_gathers(g2, 0)
            drain_gathers(1)
            process(1)
            issue_idx(g3, 1)
            return carry

        lax.fori_loop(0, nb // 2, pipe, 0)
        drain_idx(1)
        drain_gathers(0)

        plsc.subcore_barrier()
        pltpu.sync_copy(accum.at[pl.ds(s * rpt, rpt)],
                        out_hbm.at[c, pl.ds(s * rpt, rpt)])

    return k(ps, pt, ef_flat, w_flat, src, tgt)


def _sc_degree(tgt, n):
    e = tgt.shape[0]
    nc, ns = 2, 16
    nw = nc * ns
    epw = e // nw
    bsz = 80
    nb = epw // bsz
    npad = ((n + 8 * ns - 1) // (8 * ns)) * (8 * ns)
    rpt = npad // ns
    mesh = plsc.VectorSubcoreMesh(core_axis_name="c", subcore_axis_name="s")

    @functools.partial(
        pl.kernel,
        mesh=mesh,
        out_type=jax.ShapeDtypeStruct((nc, npad, 128), jnp.float32),
        scratch_types=[
            pltpu.VMEM((bsz,), jnp.int32),
            pltpu.VMEM((bsz, 128), jnp.float32),
            pltpu.VMEM_SHARED((npad, 128), jnp.float32),
        ],
    )
    def k(tgt_hbm, out_hbm, tidv, ones_rows, accum):
        c = lax.axis_index("c")
        s = lax.axis_index("s")
        wid = c * ns + s

        zero16 = jnp.zeros((LANE,), jnp.float32)
        one0 = jnp.where(lax.iota(jnp.int32, LANE) == 0, 1.0, 0.0)

        def zrow(r, carry):
            for j in range(128 // LANE):
                ones_rows[r, pl.ds(j * LANE, LANE)] = zero16
            return carry

        lax.fori_loop(0, bsz, zrow, 0)

        nfull, rem = rpt // bsz, rpt % bsz
        for zi in range(nfull):
            pltpu.sync_copy(ones_rows, accum.at[pl.ds(s * rpt + zi * bsz, bsz)])
        if rem:
            pltpu.sync_copy(ones_rows.at[pl.ds(0, rem)],
                            accum.at[pl.ds(s * rpt + nfull * bsz, rem)])

        def orow(r, carry):
            ones_rows[r, pl.ds(0, LANE)] = one0
            return carry

        lax.fori_loop(0, bsz, orow, 0)
        plsc.subcore_barrier()

        base = wid * epw

        def batch(b, carry):
            off = base + b * bsz
            pltpu.sync_copy(tgt_hbm.at[pl.ds(off, bsz)], tidv)
            pltpu.sync_copy(ones_rows, accum.at[tidv], add=True)
            return carry

        lax.fori_loop(0, nb, batch, 0)
        plsc.subcore_barrier()
        pltpu.sync_copy(accum.at[pl.ds(s * rpt, rpt)],
                        out_hbm.at[c, pl.ds(s * rpt, rpt)])

    return k(tgt)


def kernel(node_features, edge_index, edge_features, eW1, eb1, eW2, eb2,
           nW1, nb1, nW2, nb2):
    n, d = node_features.shape
    e = edge_features.shape[0]

    src = edge_index[0].astype(jnp.int32)
    tgt = edge_index[1].astype(jnp.int32)

    w_st = jnp.concatenate([eW1[:d], eW1[d:2 * d]], axis=1)
    w_flat = jnp.concatenate([eW1[2 * d:].reshape(-1), eb1])
    ef_flat = edge_features.reshape(-1)
    nw1a = nW1[:d]
    nw1b = nW1[d:]

    deg = _sc_degree(tgt, n)
    ps, pt = _tc_node_proj(node_features, w_st)
    hsum = _sc_edge_agg(ps, pt, ef_flat, w_flat, src, tgt)
    out = _tc_final(node_features, hsum[0, :n], hsum[1, :n],
                    deg[0, :n], deg[1, :n],
                    eW2, eb2.reshape(1, -1), nw1a, nw1b,
                    nb1.reshape(1, -1), nW2, nb2.reshape(1, -1))
    return out


# speedup vs baseline: 1.7293x; 1.7293x over previous
import functools

import jax
import jax.numpy as jnp
from jax import lax
from jax.experimental import pallas as pl
from jax.experimental.pallas import tpu as pltpu
from jax.experimental.pallas import tpu_sc as plsc

LANE = 16


def _tc_node_proj(x, w_st):
    n = x.shape[0]
    bn = 1000

    def body(x_ref, w_ref, ps_ref, pt_ref):
        p = jnp.dot(x_ref[...], w_ref[...], preferred_element_type=jnp.float32)
        ps_ref[...] = p[:, :128]
        pt_ref[...] = p[:, 128:]

    return pl.pallas_call(
        body,
        grid=(n // bn,),
        in_specs=[
            pl.BlockSpec((bn, 128), lambda i: (i, 0)),
            pl.BlockSpec((128, 256), lambda i: (0, 0)),
        ],
        out_specs=[
            pl.BlockSpec((bn, 128), lambda i: (i, 0)),
            pl.BlockSpec((bn, 128), lambda i: (i, 0)),
        ],
        out_shape=[jax.ShapeDtypeStruct((n, 128), jnp.float32)] * 2,
    )(x, w_st)


def _tc_final(x, agg0, agg1, deg0, deg1, eW2, eb2, nw1a, nw1b, nb1, nw2, nb2):
    n = x.shape[0]
    bn = 1000

    def body(x_ref, a0_ref, a1_ref, d0_ref, d1_ref, w2_ref, b2e_ref, wa_ref,
             wb_ref, b1_ref, w3_ref, b2_ref, out_ref):
        deg = d0_ref[...][:, :1] + d1_ref[...][:, :1]
        agg = (
            jnp.dot(a0_ref[...] + a1_ref[...], w2_ref[...],
                    preferred_element_type=jnp.float32)
            + deg * b2e_ref[...]
        )
        pre = (
            jnp.dot(x_ref[...], wa_ref[...], preferred_element_type=jnp.float32)
            + jnp.dot(agg, wb_ref[...], preferred_element_type=jnp.float32)
            + b1_ref[...]
        )
        h2 = jnp.maximum(pre, 0.0)
        out_ref[...] = (
            x_ref[...]
            + jnp.dot(h2, w3_ref[...], preferred_element_type=jnp.float32)
            + b2_ref[...]
        )

    return pl.pallas_call(
        body,
        grid=(n // bn,),
        in_specs=[
            pl.BlockSpec((bn, 128), lambda i: (i, 0)),
            pl.BlockSpec((bn, 128), lambda i: (i, 0)),
            pl.BlockSpec((bn, 128), lambda i: (i, 0)),
            pl.BlockSpec((bn, 128), lambda i: (i, 0)),
            pl.BlockSpec((bn, 128), lambda i: (i, 0)),
            pl.BlockSpec((128, 128), lambda i: (0, 0)),
            pl.BlockSpec((1, 128), lambda i: (0, 0)),
            pl.BlockSpec((128, 128), lambda i: (0, 0)),
            pl.BlockSpec((128, 128), lambda i: (0, 0)),
            pl.BlockSpec((1, 128), lambda i: (0, 0)),
            pl.BlockSpec((128, 128), lambda i: (0, 0)),
            pl.BlockSpec((1, 128), lambda i: (0, 0)),
        ],
        out_specs=pl.BlockSpec((bn, 128), lambda i: (i, 0)),
        out_shape=jax.ShapeDtypeStruct((n, 128), jnp.float32),
    )(x, agg0, agg1, deg0, deg1, eW2, eb2, nw1a, nw1b, nb1, nw2, nb2)


def _sc_edge_agg(ps, pt, ef_flat, w_flat, src, tgt):
    n = ps.shape[0]
    e = src.shape[0]
    nc, ns = 2, 16
    nw = nc * ns
    epw = e // nw
    bsz = 40
    nb = epw // bsz
    npad = ((n + 8 * ns - 1) // (8 * ns)) * (8 * ns)
    rpt = npad // ns
    mesh = plsc.VectorSubcoreMesh(core_axis_name="c", subcore_axis_name="s")

    @functools.partial(
        pl.kernel,
        mesh=mesh,
        out_type=jax.ShapeDtypeStruct((nc, npad, 128), jnp.float32),
        scratch_types=[
            pltpu.VMEM((bsz,), jnp.int32),
            pltpu.VMEM((bsz,), jnp.int32),
            pltpu.VMEM((bsz,), jnp.int32),
            pltpu.VMEM((bsz,), jnp.int32),
            pltpu.VMEM((bsz, 128), jnp.float32),
            pltpu.VMEM((bsz, 128), jnp.float32),
            pltpu.VMEM((bsz, 128), jnp.float32),
            pltpu.VMEM((bsz, 128), jnp.float32),
            pltpu.VMEM((4 * bsz + LANE,), jnp.float32),
            pltpu.VMEM((4 * bsz + LANE,), jnp.float32),
            pltpu.VMEM((640,), jnp.float32),
            pltpu.VMEM((bsz, 128), jnp.float32),
            pltpu.SemaphoreType.DMA,
            pltpu.SemaphoreType.DMA,
            pltpu.SemaphoreType.DMA,
            pltpu.SemaphoreType.DMA,
            pltpu.VMEM_SHARED((npad, 128), jnp.float32),
        ],
    )
    def k(ps_hbm, pt_hbm, ef_hbm, w_hbm, src_hbm, tgt_hbm, out_hbm,
          sid0, sid1, tid0, tid1, ps0, ps1, pt0, pt1, ef0, ef1, wtab, hv,
          semi0, semi1, semg0, semg1, accum):
        c = lax.axis_index("c")
        s = lax.axis_index("s")
        wid = c * ns + s
        sidb = (sid0, sid1)
        tidb = (tid0, tid1)
        psb = (ps0, ps1)
        ptb = (pt0, pt1)
        efb = (ef0, ef1)
        semi = (semi0, semi1)
        semg = (semg0, semg1)

        zero16 = jnp.zeros((LANE,), jnp.float32)

        def zrow(r, carry):
            for j in range(128 // LANE):
                ps0[r, pl.ds(j * LANE, LANE)] = zero16
            return carry

        lax.fori_loop(0, bsz, zrow, 0)
        pltpu.sync_copy(w_hbm, wtab)

        nfull, rem = rpt // bsz, rpt % bsz
        for zi in range(nfull):
            pltpu.sync_copy(ps0, accum.at[pl.ds(s * rpt + zi * bsz, bsz)])
        if rem:
            pltpu.sync_copy(ps0.at[pl.ds(0, rem)],
                            accum.at[pl.ds(s * rpt + nfull * bsz, rem)])
        plsc.subcore_barrier()

        base = wid * epw

        def issue_idx(g, p):
            off = base + g * bsz
            pltpu.async_copy(src_hbm.at[pl.ds(off, bsz)], sidb[p], semi[p])
            pltpu.async_copy(tgt_hbm.at[pl.ds(off, bsz)], tidb[p], semi[p])

        def drain_idx(p):
            pltpu.make_async_copy(src_hbm.at[pl.ds(0, bsz)], sidb[p],
                                  semi[p]).wait()
            pltpu.make_async_copy(tgt_hbm.at[pl.ds(0, bsz)], tidb[p],
                                  semi[p]).wait()

        def issue_gathers(g, p):
            off = base + g * bsz
            pltpu.async_copy(ps_hbm.at[sidb[p]], psb[p], semg[p])
            pltpu.async_copy(pt_hbm.at[tidb[p]], ptb[p], semg[p])
            pltpu.async_copy(ef_hbm.at[pl.ds(4 * off, 4 * bsz)],
                             efb[p].at[pl.ds(0, 4 * bsz)], semg[p])

        def drain_gathers(p):
            for dst in (psb[p], ptb[p]):
                pltpu.make_async_copy(ps_hbm.at[pl.ds(0, bsz)], dst,
                                      semg[p]).wait()
            pltpu.make_async_copy(ef_hbm.at[pl.ds(0, 4 * bsz)],
                                  efb[p].at[pl.ds(0, 4 * bsz)],
                                  semg[p]).wait()

        bidx = [jnp.full((LANE,), kk, jnp.int32) for kk in range(4)]

        def process(p):
            wv = [[wtab[pl.ds(kk * 128 + j * LANE, LANE)]
                   for j in range(128 // LANE)] for kk in range(4)]
            ebv = [wtab[pl.ds(512 + j * LANE, LANE)]
                   for j in range(128 // LANE)]

            def row(r, rc):
                efc = efb[p][pl.ds(4 * r, LANE)]
                eb = [efc.at[bidx[kk]].get(mode="promise_in_bounds")
                      for kk in range(4)]
                for j in range(128 // LANE):
                    sl = pl.ds(j * LANE, LANE)
                    v = psb[p][r, sl] + ptb[p][r, sl] + ebv[j]
                    for kk in range(4):
                        v = v + eb[kk] * wv[kk][j]
                    hv[r, sl] = jnp.maximum(v, 0.0)
                return rc

            lax.fori_loop(0, bsz, row, 0)
            pltpu.sync_copy(hv, accum.at[tidb[p]], add=True)

        pltpu.sync_copy(src_hbm.at[pl.ds(base, bsz)], sid0)
        pltpu.sync_copy(tgt_hbm.at[pl.ds(base, bsz)], tid0)
        issue_gathers(0, 0)
        issue_idx(1, 1)

        def pipe(i, carry):
            g1 = 2 * i + 1
            g2 = jnp.minimum(2 * i + 2, nb - 1)
            g3 = jnp.minimum(2 * i + 3, nb - 1)
            drain_idx(1)
            issue_gathers(g1, 1)
            drain_gathers(0)
            process(0)
            issue_idx(g2, 0)
            drain_idx(0)
            issue_gathers(g2, 0)
            drain_gathers(1)
            process(1)
            issue_idx(g3, 1)
            return carry

        lax.fori_loop(0, nb // 2, pipe, 0)
        drain_idx(1)
        drain_gathers(0)

        plsc.subcore_barrier()
        pltpu.sync_copy(accum.at[pl.ds(s * rpt, rpt)],
                        out_hbm.at[c, pl.ds(s * rpt, rpt)])

    return k(ps, pt, ef_flat, w_flat, src, tgt)


def _sc_degree(tgt, n):
    e = tgt.shape[0]
    nc, ns = 2, 16
    nw = nc * ns
    epw = e // nw
    bsz = 80
    nb = epw // bsz
    npad = ((n + 8 * ns - 1) // (8 * ns)) * (8 * ns)
    rpt = npad // ns
    mesh = plsc.VectorSubcoreMesh(core_axis_name="c", subcore_axis_name="s")

    @functools.partial(
        pl.kernel,
        mesh=mesh,
        out_type=jax.ShapeDtypeStruct((nc, npad, 128), jnp.float32),
        scratch_types=[
            pltpu.VMEM((bsz,), jnp.int32),
            pltpu.VMEM((bsz, 128), jnp.float32),
            pltpu.VMEM_SHARED((npad, 128), jnp.float32),
        ],
    )
    def k(tgt_hbm, out_hbm, tidv, ones_rows, accum):
        c = lax.axis_index("c")
        s = lax.axis_index("s")
        wid = c * ns + s

        zero16 = jnp.zeros((LANE,), jnp.float32)
        one0 = jnp.where(lax.iota(jnp.int32, LANE) == 0, 1.0, 0.0)

        def zrow(r, carry):
            for j in range(128 // LANE):
                ones_rows[r, pl.ds(j * LANE, LANE)] = zero16
            return carry

        lax.fori_loop(0, bsz, zrow, 0)

        nfull, rem = rpt // bsz, rpt % bsz
        for zi in range(nfull):
            pltpu.sync_copy(ones_rows, accum.at[pl.ds(s * rpt + zi * bsz, bsz)])
        if rem:
            pltpu.sync_copy(ones_rows.at[pl.ds(0, rem)],
                            accum.at[pl.ds(s * rpt + nfull * bsz, rem)])

        def orow(r, carry):
            ones_rows[r, pl.ds(0, LANE)] = one0
            return carry

        lax.fori_loop(0, bsz, orow, 0)
        plsc.subcore_barrier()

        base = wid * epw

        def batch(b, carry):
            off = base + b * bsz
            pltpu.sync_copy(tgt_hbm.at[pl.ds(off, bsz)], tidv)
            pltpu.sync_copy(ones_rows, accum.at[tidv], add=True)
            return carry

        lax.fori_loop(0, nb, batch, 0)
        plsc.subcore_barrier()
        pltpu.sync_copy(accum.at[pl.ds(s * rpt, rpt)],
                        out_hbm.at[c, pl.ds(s * rpt, rpt)])

    return k(tgt)


def kernel(node_features, edge_index, edge_features, eW1, eb1, eW2, eb2,
           nW1, nb1, nW2, nb2):
    n, d = node_features.shape
    e = edge_features.shape[0]

    src = edge_index[0].astype(jnp.int32)
    tgt = edge_index[1].astype(jnp.int32)

    w_st = jnp.concatenate([eW1[:d], eW1[d:2 * d]], axis=1)
    w_flat = jnp.concatenate([eW1[2 * d:].reshape(-1), eb1])
    ef_flat = edge_features.reshape(-1)
    nw1a = nW1[:d]
    nw1b = nW1[d:]

    deg = _sc_degree(tgt, n)
    ps, pt = _tc_node_proj(node_features, w_st)
    hsum = _sc_edge_agg(ps, pt, ef_flat, w_flat, src, tgt)
    out = _tc_final(node_features, hsum[0, :n], hsum[1, :n],
                    deg[0, :n], deg[1, :n],
                    eW2, eb2.reshape(1, -1), nw1a, nw1b,
                    nb1.reshape(1, -1), nW2, nb2.reshape(1, -1))
    return out
